# baseline (speedup 1.0000x reference)
import jax
import jax.numpy as jnp
from jax.experimental import pallas as pl
from jax.experimental.pallas import tpu as pltpu

_RANK = 64
_SCALE = 1.0 / _RANK
_TILE = 512


def _body(ids_ref, x_ref, b_ref, a_ref, o_ref):
    xb = x_ref[...].astype(jnp.bfloat16)
    t = jnp.dot(xb, b_ref[0], preferred_element_type=jnp.float32)
    o_ref[...] = jnp.dot(t.astype(jnp.bfloat16), a_ref[...],
                         preferred_element_type=jnp.float32)


def kernel(x, A, B, adapter_ids):
    Bt, S, H = x.shape
    R, OUT = A.shape
    n_adapters = R // _RANK
    B3 = jnp.transpose(B.reshape(H, n_adapters, _RANK), (1, 0, 2))
    B3 = B3.astype(jnp.bfloat16)
    A = (A * _SCALE).astype(jnp.bfloat16)
    x2 = x.reshape(Bt * S, H)
    tiles_per_batch = S // _TILE
    grid = (Bt * S // _TILE,)
    grid_spec = pltpu.PrefetchScalarGridSpec(
        num_scalar_prefetch=1,
        grid=grid,
        in_specs=[
            pl.BlockSpec((_TILE, H), lambda i, ids: (i, 0)),
            pl.BlockSpec((1, H, _RANK),
                         lambda i, ids: (ids[i // tiles_per_batch], 0, 0)),
            pl.BlockSpec((_RANK, OUT),
                         lambda i, ids: (ids[i // tiles_per_batch], 0)),
        ],
        out_specs=pl.BlockSpec((_TILE, OUT), lambda i, ids: (i, 0)),
    )
    out = pl.pallas_call(
        _body,
        grid_spec=grid_spec,
        out_shape=jax.ShapeDtypeStruct((Bt * S, OUT), jnp.float32),
        compiler_params=pltpu.CompilerParams(
            dimension_semantics=("arbitrary",)),
    )(adapter_ids, x2, B3, A)
    return out.reshape(Bt, S, OUT)


# speedup vs baseline: 1.0014x; 1.0014x over previous
import jax
import jax.numpy as jnp
from jax.experimental import pallas as pl
from jax.experimental.pallas import tpu as pltpu

_RANK = 64
_SCALE = 1.0 / _RANK
_TILE = 512


def _body(ids_ref, x_ref, b_ref, a_ref, o_ref):
    xb = x_ref[...].astype(jnp.bfloat16)
    t = jnp.dot(xb, b_ref[0], preferred_element_type=jnp.float32)
    o_ref[...] = jnp.dot(t.astype(jnp.bfloat16), a_ref[...],
                         preferred_element_type=jnp.float32)


def kernel(x, A, B, adapter_ids):
    Bt, S, H = x.shape
    R, OUT = A.shape
    n_adapters = R // _RANK
    B3 = jnp.transpose(B.reshape(H, n_adapters, _RANK), (1, 0, 2))
    B3 = B3.astype(jnp.bfloat16)
    A = (A * _SCALE).astype(jnp.bfloat16)
    x2 = x.reshape(Bt * S, H)
    tiles_per_batch = S // _TILE
    grid = (Bt * S // _TILE,)
    grid_spec = pltpu.PrefetchScalarGridSpec(
        num_scalar_prefetch=1,
        grid=grid,
        in_specs=[
            pl.BlockSpec((_TILE, H), lambda i, ids: (i, 0)),
            pl.BlockSpec((1, H, _RANK),
                         lambda i, ids: (ids[i // tiles_per_batch], 0, 0)),
            pl.BlockSpec((_RANK, OUT),
                         lambda i, ids: (ids[i // tiles_per_batch], 0)),
        ],
        out_specs=pl.BlockSpec((_TILE, OUT), lambda i, ids: (i, 0)),
    )
    out = pl.pallas_call(
        _body,
        grid_spec=grid_spec,
        out_shape=jax.ShapeDtypeStruct((Bt * S, OUT), jnp.float32),
        compiler_params=pltpu.CompilerParams(
            dimension_semantics=("parallel",)),
    )(adapter_ids, x2, B3, A)
    return out.reshape(Bt, S, OUT)


# speedup vs baseline: 1.0803x; 1.0788x over previous
import jax
import jax.numpy as jnp
from jax.experimental import pallas as pl
from jax.experimental.pallas import tpu as pltpu

_RANK = 64
_SCALE = 1.0 / _RANK
_TILE = 512


_CHUNK = 256


def _body(ids_ref, x_ref, b_ref, a_ref, o_ref):
    for c in range(_TILE // _CHUNK):
        rows = pl.ds(c * _CHUNK, _CHUNK)
        xb = x_ref[rows, :].astype(jnp.bfloat16)
        t = jnp.dot(xb, b_ref[0], preferred_element_type=jnp.float32)
        o_ref[rows, :] = jnp.dot(t.astype(jnp.bfloat16), a_ref[...],
                                 preferred_element_type=jnp.float32)


def kernel(x, A, B, adapter_ids):
    Bt, S, H = x.shape
    R, OUT = A.shape
    n_adapters = R // _RANK
    B3 = jnp.transpose(B.reshape(H, n_adapters, _RANK), (1, 0, 2))
    B3 = B3.astype(jnp.bfloat16)
    A = (A * _SCALE).astype(jnp.bfloat16)
    x2 = x.reshape(Bt * S, H)
    tiles_per_batch = S // _TILE
    grid = (Bt * S // _TILE,)
    grid_spec = pltpu.PrefetchScalarGridSpec(
        num_scalar_prefetch=1,
        grid=grid,
        in_specs=[
            pl.BlockSpec((_TILE, H), lambda i, ids: (i, 0)),
            pl.BlockSpec((1, H, _RANK),
                         lambda i, ids: (ids[i // tiles_per_batch], 0, 0)),
            pl.BlockSpec((_RANK, OUT),
                         lambda i, ids: (ids[i // tiles_per_batch], 0)),
        ],
        out_specs=pl.BlockSpec((_TILE, OUT), lambda i, ids: (i, 0)),
    )
    out = pl.pallas_call(
        _body,
        grid_spec=grid_spec,
        out_shape=jax.ShapeDtypeStruct((Bt * S, OUT), jnp.float32),
        compiler_params=pltpu.CompilerParams(
            dimension_semantics=("parallel",)),
    )(adapter_ids, x2, B3, A)
    return out.reshape(Bt, S, OUT)
